# Initial kernel scaffold; baseline (speedup 1.0000x reference)
import jax
import jax.numpy as jnp
from jax.experimental import pallas as pl


def kernel(x, edge_attr, w, src_idx, dst_idx):
    raise NotImplementedError("write your pallas kernel here")


# speedup vs baseline: 2.5903x; 2.5903x over previous
import functools

import jax
import jax.numpy as jnp
from jax import lax
from jax.experimental import pallas as pl
from jax.experimental.pallas import tpu as pltpu
from jax.experimental.pallas import tpu_sc as plsc

N_NODES = 10000
N_EDGES = 320000
D_FEAT = 128
N_SEG = 16
SEG_W = 8

NC = 2
NS = 16
NW = NC * NS
EPW = N_EDGES // NW
CH = 80
NCH = EPW // CH
ROW_CH = 80
NRC = N_NODES // ROW_CH

GATE_BLK = 4000


def _gate_body(ea_ref, wb_ref, gate_ref):
    gate_ref[...] = jnp.dot(ea_ref[...], wb_ref[...],
                            preferred_element_type=jnp.float32)


def _gate_matmul(ea_flat, w_block):
    grid = (N_EDGES // SEG_W) // GATE_BLK
    return pl.pallas_call(
        _gate_body,
        grid=(grid,),
        in_specs=[
            pl.BlockSpec((GATE_BLK, 128), lambda i: (i, 0)),
            pl.BlockSpec((128, 128), lambda i: (0, 0)),
        ],
        out_specs=pl.BlockSpec((GATE_BLK, 128), lambda i: (i, 0)),
        out_shape=jax.ShapeDtypeStruct((N_EDGES // SEG_W, 128), jnp.float32),
    )(ea_flat, w_block)


def _sc_body(xp_hbm, gate_hbm, src_hbm, dst_hbm, out_hbm,
             src_v, dst_v, gate_v, msg_v, zero_v, acc_sh, sem):
    c = lax.axis_index("c")
    s = lax.axis_index("s")
    wid = s * NC + c
    base = wid * EPW

    def _zrow(e, carry):
        for k in range(D_FEAT // 16):
            zero_v[e, pl.ds(k * 16, 16)] = jnp.zeros((16,), jnp.float32)
        return carry
    lax.fori_loop(0, ROW_CH, _zrow, 0)
    for t in range((NRC + NS - 1) // NS):
        j = t * NS + s
        @pl.when(j < NRC)
        def _():
            pltpu.sync_copy(zero_v, acc_sh.at[pl.ds(j * ROW_CH, ROW_CH), :])
    plsc.subcore_barrier()

    def _chunk(i, carry):
        off = base + i * CH
        pltpu.sync_copy(src_hbm.at[pl.ds(off, CH)], src_v)
        pltpu.sync_copy(dst_hbm.at[pl.ds(off, CH)], dst_v)
        pltpu.sync_copy(gate_hbm.at[pl.ds(off, CH), :], gate_v)
        pltpu.async_copy(xp_hbm.at[src_v], msg_v, sem).wait()

        def _edge(e, ecarry):
            g = gate_v[e, :]
            for k in range(D_FEAT // 16):
                msg_v[e, pl.ds(k * 16, 16)] = msg_v[e, pl.ds(k * 16, 16)] * g
            return ecarry
        lax.fori_loop(0, CH, _edge, 0)

        pltpu.sync_copy(msg_v, acc_sh.at[dst_v], add=True)
        return carry
    lax.fori_loop(0, NCH, _chunk, 0)

    plsc.subcore_barrier()
    for t in range((NRC + NS - 1) // NS):
        j = t * NS + s
        @pl.when(j < NRC)
        def _():
            pltpu.sync_copy(acc_sh.at[pl.ds(j * ROW_CH, ROW_CH), :],
                            out_hbm.at[c, pl.ds(j * ROW_CH, ROW_CH), :])


def _sc_scatter(xp, gate, src_idx, dst_idx):
    mesh = plsc.VectorSubcoreMesh(core_axis_name="c", subcore_axis_name="s")
    kern = functools.partial(
        pl.kernel,
        mesh=mesh,
        out_type=jax.ShapeDtypeStruct((NC, N_NODES, D_FEAT), jnp.float32),
        scratch_types=[
            pltpu.VMEM((CH,), jnp.int32),
            pltpu.VMEM((CH,), jnp.int32),
            pltpu.VMEM((CH, N_SEG), jnp.float32),
            pltpu.VMEM((CH, D_FEAT), jnp.float32),
            pltpu.VMEM((ROW_CH, D_FEAT), jnp.float32),
            pltpu.VMEM_SHARED((N_NODES, D_FEAT), jnp.float32),
            pltpu.SemaphoreType.DMA,
        ],
    )(_sc_body)
    return kern(xp, gate, src_idx, dst_idx)


def _combine_body(p_ref, out_ref):
    out_ref[...] = p_ref[0] + p_ref[1]


def _combine(partials):
    return pl.pallas_call(
        _combine_body,
        grid=(5,),
        in_specs=[pl.BlockSpec((NC, 2000, D_FEAT), lambda i: (0, i, 0))],
        out_specs=pl.BlockSpec((2000, D_FEAT), lambda i: (i, 0)),
        out_shape=jax.ShapeDtypeStruct((N_NODES, D_FEAT), jnp.float32),
    )(partials)


def kernel(x, edge_attr, w, src_idx, dst_idx):
    w_block = jnp.kron(jnp.eye(SEG_W, dtype=w.dtype), w)
    ea_flat = edge_attr.reshape(N_EDGES // SEG_W, SEG_W * N_SEG)
    gate = _gate_matmul(ea_flat, w_block).reshape(N_EDGES, N_SEG)

    xp = x.reshape(N_NODES, N_SEG, SEG_W).transpose(0, 2, 1)
    xp = xp.reshape(N_NODES, D_FEAT)

    partials = _sc_scatter(xp, gate, src_idx, dst_idx)
    outp = _combine(partials)

    out = outp.reshape(N_NODES, SEG_W, N_SEG).transpose(0, 2, 1)
    return out.reshape(N_NODES, D_FEAT)


# speedup vs baseline: 3.9828x; 1.5376x over previous
import functools

import jax
import jax.numpy as jnp
from jax import lax
from jax.experimental import pallas as pl
from jax.experimental.pallas import tpu as pltpu
from jax.experimental.pallas import tpu_sc as plsc

N_NODES = 10000
N_EDGES = 320000
D_FEAT = 128
N_SEG = 16
SEG_W = 8

NC = 2
NS = 16
NW = NC * NS
EPW = N_EDGES // NW
CH = 40
NCH = EPW // CH
ROW_CH = 40
NRC = N_NODES // ROW_CH

GATE_BLK = 4000


def _gate_body(ea_ref, wb_ref, gate_ref):
    gate_ref[...] = jnp.dot(ea_ref[...], wb_ref[...],
                            preferred_element_type=jnp.float32)


def _gate_matmul(ea_flat, w_block):
    grid = (N_EDGES // SEG_W) // GATE_BLK
    return pl.pallas_call(
        _gate_body,
        grid=(grid,),
        in_specs=[
            pl.BlockSpec((GATE_BLK, 128), lambda i: (i, 0)),
            pl.BlockSpec((128, 128), lambda i: (0, 0)),
        ],
        out_specs=pl.BlockSpec((GATE_BLK, 128), lambda i: (i, 0)),
        out_shape=jax.ShapeDtypeStruct((N_EDGES // SEG_W, 128), jnp.float32),
    )(ea_flat, w_block)


RB = 4
N_MAIN = (NCH // RB) * RB
N_TAIL = NCH - N_MAIN


def _sc_body(xp_hbm, gate_hbm, sd_hbm, out_hbm, *refs):
    idx_vs = refs[0:RB]
    gate_vs = refs[RB:2 * RB]
    msg_vs = refs[2 * RB:3 * RB]
    acc_sh = refs[3 * RB]
    sem_i = refs[3 * RB + 1:4 * RB + 1]
    sem_g = refs[4 * RB + 1:5 * RB + 1]
    sem_t = refs[5 * RB + 1:6 * RB + 1]
    sem_s = refs[6 * RB + 1:7 * RB + 1]

    c = lax.axis_index("c")
    s = lax.axis_index("s")
    wid = s * NC + c

    def _start_idx(r, p):
        pltpu.async_copy(sd_hbm.at[wid, p], idx_vs[r], sem_i[r])

    def _wait_idx(r):
        pltpu.make_async_copy(sd_hbm.at[0, 0], idx_vs[r], sem_i[r]).wait()

    def _start_fetch(r, p):
        pltpu.async_copy(xp_hbm.at[idx_vs[r].at[0]], msg_vs[r], sem_g[r])
        pltpu.async_copy(gate_hbm.at[wid, p], gate_vs[r], sem_t[r])

    def _wait_fetch(r):
        pltpu.make_async_copy(xp_hbm.at[idx_vs[r].at[0]], msg_vs[r],
                              sem_g[r]).wait()
        pltpu.make_async_copy(gate_hbm.at[0, 0], gate_vs[r], sem_t[r]).wait()

    def _start_scatter(r):
        pltpu.async_copy(msg_vs[r], acc_sh.at[idx_vs[r].at[1]], sem_s[r],
                         add=True)

    def _wait_scatter(r):
        pltpu.make_async_copy(msg_vs[r], acc_sh.at[idx_vs[r].at[1]],
                              sem_s[r]).wait()

    zero_v = msg_vs[0]
    def _zrow(e, carry):
        for k in range(D_FEAT // 16):
            zero_v[e, pl.ds(k * 16, 16)] = jnp.zeros((16,), jnp.float32)
        return carry
    lax.fori_loop(0, CH, _zrow, 0)
    for t in range((NRC + NS - 1) // NS):
        j = t * NS + s
        @pl.when(j < NRC)
        def _():
            pltpu.sync_copy(zero_v, acc_sh.at[pl.ds(j * ROW_CH, ROW_CH), :])
    plsc.subcore_barrier()

    for r in range(RB - 1):
        _start_idx(r, r)
    for r in range(RB - 2):
        _wait_idx(r)
        _start_fetch(r, r)

    def _process(r):
        _wait_fetch(r)

        def _edge(e, ecarry):
            g = gate_vs[r][e, :]
            for k in range(D_FEAT // 16):
                msg_vs[r][e, pl.ds(k * 16, 16)] = (
                    msg_vs[r][e, pl.ds(k * 16, 16)] * g)
            return ecarry
        lax.fori_loop(0, CH, _edge, 0)

        _start_scatter(r)

    def _round(i0, carry):
        for r in range(RB):
            i = i0 * RB + r
            _process(r)

            pa = i + RB - 1
            ra = (r + RB - 1) % RB
            @pl.when(jnp.logical_and(pa < NCH, pa >= RB))
            def _():
                _wait_scatter(ra)
            @pl.when(pa < NCH)
            def _():
                _start_idx(ra, pa)

            pb = i + RB - 2
            rb = (r + RB - 2) % RB
            @pl.when(jnp.logical_and(pb < NCH, pb >= RB - 2))
            def _():
                _wait_idx(rb)
                _start_fetch(rb, pb)
        return carry
    lax.fori_loop(0, N_MAIN // RB, _round, 0)
    for t in range(N_TAIL):
        _process((N_MAIN + t) % RB)

    for r in range(RB):
        _wait_scatter(r)
    plsc.subcore_barrier()
    for t in range((NRC + NS - 1) // NS):
        j = t * NS + s
        @pl.when(j < NRC)
        def _():
            pltpu.sync_copy(acc_sh.at[pl.ds(j * ROW_CH, ROW_CH), :],
                            out_hbm.at[c, pl.ds(j * ROW_CH, ROW_CH), :])


def _sc_scatter(xp, gate4, sd):
    mesh = plsc.VectorSubcoreMesh(core_axis_name="c", subcore_axis_name="s")
    kern = functools.partial(
        pl.kernel,
        mesh=mesh,
        out_type=jax.ShapeDtypeStruct((NC, N_NODES, D_FEAT), jnp.float32),
        scratch_types=(
            [pltpu.VMEM((2, CH), jnp.int32) for _ in range(RB)]
            + [pltpu.VMEM((CH, N_SEG), jnp.float32) for _ in range(RB)]
            + [pltpu.VMEM((CH, D_FEAT), jnp.float32) for _ in range(RB)]
            + [pltpu.VMEM_SHARED((N_NODES, D_FEAT), jnp.float32)]
            + [pltpu.SemaphoreType.DMA for _ in range(4 * RB)]
        ),
    )(_sc_body)
    return kern(xp, gate4, sd)


def _combine_body(p_ref, out_ref):
    out_ref[...] = p_ref[0] + p_ref[1]


def _combine(partials):
    return pl.pallas_call(
        _combine_body,
        grid=(5,),
        in_specs=[pl.BlockSpec((NC, 2000, D_FEAT), lambda i: (0, i, 0))],
        out_specs=pl.BlockSpec((2000, D_FEAT), lambda i: (i, 0)),
        out_shape=jax.ShapeDtypeStruct((N_NODES, D_FEAT), jnp.float32),
    )(partials)


def kernel(x, edge_attr, w, src_idx, dst_idx):
    w_block = jnp.kron(jnp.eye(SEG_W, dtype=w.dtype), w)
    ea_flat = edge_attr.reshape(N_EDGES // SEG_W, SEG_W * N_SEG)
    gate = _gate_matmul(ea_flat, w_block).reshape(N_EDGES, N_SEG)

    xp = x.reshape(N_NODES, N_SEG, SEG_W).transpose(0, 2, 1)
    xp = xp.reshape(N_NODES, D_FEAT)

    gate4 = gate.reshape(NW, NCH, CH, N_SEG)
    sd = jnp.stack([src_idx.reshape(NW, NCH, CH),
                    dst_idx.reshape(NW, NCH, CH)], axis=2)
    partials = _sc_scatter(xp, gate4, sd)
    outp = _combine(partials)

    out = outp.reshape(N_NODES, SEG_W, N_SEG).transpose(0, 2, 1)
    return out.reshape(N_NODES, D_FEAT)


# speedup vs baseline: 3.9846x; 1.0005x over previous
import functools

import jax
import jax.numpy as jnp
from jax import lax
from jax.experimental import pallas as pl
from jax.experimental.pallas import tpu as pltpu
from jax.experimental.pallas import tpu_sc as plsc

N_NODES = 10000
N_EDGES = 320000
D_FEAT = 128
N_SEG = 16
SEG_W = 8

NC = 2
NS = 16
NW = NC * NS
EPW = N_EDGES // NW
CH = 40
NCH = EPW // CH
ROW_CH = 40
NRC = N_NODES // ROW_CH

GATE_BLK = 4000

RB = 4
N_MAIN = (NCH // RB) * RB
N_TAIL = NCH - N_MAIN


def _gate_body(ea_ref, wb_ref, gate_ref):
    gate_ref[...] = jnp.dot(ea_ref[...], wb_ref[...],
                            preferred_element_type=jnp.float32)


def _gate_matmul(ea_flat, w_block):
    grid = (N_EDGES // SEG_W) // GATE_BLK
    return pl.pallas_call(
        _gate_body,
        grid=(grid,),
        in_specs=[
            pl.BlockSpec((GATE_BLK, 128), lambda i: (i, 0)),
            pl.BlockSpec((128, 128), lambda i: (0, 0)),
        ],
        out_specs=pl.BlockSpec((GATE_BLK, 128), lambda i: (i, 0)),
        out_shape=jax.ShapeDtypeStruct((N_EDGES // SEG_W, 128), jnp.float32),
    )(ea_flat, w_block)


def _sc_body(xp_hbm, gate_hbm, sd_hbm, out_hbm, *refs):
    idx_vs = refs[0:RB]
    gate_vs = refs[RB:2 * RB]
    msg_vs = refs[2 * RB:3 * RB]
    acc_sh = refs[3 * RB]
    sem_i = refs[3 * RB + 1:4 * RB + 1]
    sem_g = refs[4 * RB + 1:5 * RB + 1]
    sem_t = refs[5 * RB + 1:6 * RB + 1]
    sem_s = refs[6 * RB + 1:7 * RB + 1]

    c = lax.axis_index("c")
    s = lax.axis_index("s")
    wid = s * NC + c

    def _start_idx(r, p):
        pltpu.async_copy(sd_hbm.at[wid, p], idx_vs[r], sem_i[r])

    def _wait_idx(r):
        pltpu.make_async_copy(sd_hbm.at[0, 0], idx_vs[r], sem_i[r]).wait()

    def _start_fetch(r, p):
        pltpu.async_copy(xp_hbm.at[idx_vs[r].at[0]], msg_vs[r], sem_g[r])
        pltpu.async_copy(gate_hbm.at[pl.ds((wid * NCH + p) * CH, CH), :],
                         gate_vs[r], sem_t[r])

    def _wait_fetch(r):
        pltpu.make_async_copy(xp_hbm.at[idx_vs[r].at[0]], msg_vs[r],
                              sem_g[r]).wait()
        pltpu.make_async_copy(gate_hbm.at[pl.ds(0, CH), :], gate_vs[r],
                              sem_t[r]).wait()

    def _start_scatter(r):
        pltpu.async_copy(msg_vs[r], acc_sh.at[idx_vs[r].at[1]], sem_s[r],
                         add=True)

    def _wait_scatter(r):
        pltpu.make_async_copy(msg_vs[r], acc_sh.at[idx_vs[r].at[1]],
                              sem_s[r]).wait()

    zero_v = msg_vs[0]
    def _zrow(e, carry):
        for k in range(D_FEAT // 16):
            zero_v[e, pl.ds(k * 16, 16)] = jnp.zeros((16,), jnp.float32)
        return carry
    lax.fori_loop(0, ROW_CH, _zrow, 0)
    for t in range((NRC + NS - 1) // NS):
        j = t * NS + s
        @pl.when(j < NRC)
        def _():
            pltpu.sync_copy(zero_v, acc_sh.at[pl.ds(j * ROW_CH, ROW_CH), :])
    plsc.subcore_barrier()

    def _process(r):
        _wait_fetch(r)

        def _edge(e, ecarry):
            g = gate_vs[r][e, :]
            for k in range(D_FEAT // 16):
                msg_vs[r][e, pl.ds(k * 16, 16)] = (
                    msg_vs[r][e, pl.ds(k * 16, 16)] * g)
            return ecarry
        lax.fori_loop(0, CH, _edge, 0)

        _start_scatter(r)

    for r in range(RB - 1):
        _start_idx(r, r)
    for r in range(RB - 2):
        _wait_idx(r)
        _start_fetch(r, r)

    def _round(i0, carry):
        for r in range(RB):
            i = i0 * RB + r
            _process(r)

            pa = i + RB - 1
            ra = (r + RB - 1) % RB
            @pl.when(jnp.logical_and(pa < NCH, pa >= RB))
            def _():
                _wait_scatter(ra)
            @pl.when(pa < NCH)
            def _():
                _start_idx(ra, pa)

            pb = i + RB - 2
            rb = (r + RB - 2) % RB
            @pl.when(jnp.logical_and(pb < NCH, pb >= RB - 2))
            def _():
                _wait_idx(rb)
                _start_fetch(rb, pb)
        return carry
    lax.fori_loop(0, N_MAIN // RB, _round, 0)
    for t in range(N_TAIL):
        _process((N_MAIN + t) % RB)

    for r in range(RB):
        _wait_scatter(r)
    plsc.subcore_barrier()
    for t in range((NRC + NS - 1) // NS):
        j = t * NS + s
        @pl.when(j < NRC)
        def _():
            pltpu.sync_copy(acc_sh.at[pl.ds(j * ROW_CH, ROW_CH), :],
                            out_hbm.at[c, pl.ds(j * ROW_CH, ROW_CH), :])


def _sc_scatter(xp, gate4, sd):
    mesh = plsc.VectorSubcoreMesh(core_axis_name="c", subcore_axis_name="s")
    kern = functools.partial(
        pl.kernel,
        mesh=mesh,
        out_type=jax.ShapeDtypeStruct((NC, N_NODES, D_FEAT), jnp.float32),
        scratch_types=(
            [pltpu.VMEM((2, CH), jnp.int32) for _ in range(RB)]
            + [pltpu.VMEM((CH, N_SEG), jnp.float32) for _ in range(RB)]
            + [pltpu.VMEM((CH, D_FEAT), jnp.float32) for _ in range(RB)]
            + [pltpu.VMEM_SHARED((N_NODES, D_FEAT), jnp.float32)]
            + [pltpu.SemaphoreType.DMA for _ in range(4 * RB)]
        ),
    )(_sc_body)
    return kern(xp, gate4, sd)


def _combine_body(p_ref, out_ref):
    out_ref[...] = p_ref[0] + p_ref[1]


def _combine(partials):
    return pl.pallas_call(
        _combine_body,
        grid=(5,),
        in_specs=[pl.BlockSpec((NC, 2000, D_FEAT), lambda i: (0, i, 0))],
        out_specs=pl.BlockSpec((2000, D_FEAT), lambda i: (i, 0)),
        out_shape=jax.ShapeDtypeStruct((N_NODES, D_FEAT), jnp.float32),
    )(partials)


def kernel(x, edge_attr, w, src_idx, dst_idx):
    w_block = jnp.kron(jnp.eye(SEG_W, dtype=w.dtype), w)
    ea_flat = edge_attr.reshape(N_EDGES // SEG_W, SEG_W * N_SEG)
    gate = _gate_matmul(ea_flat, w_block).reshape(N_EDGES, N_SEG)

    xp = x.reshape(N_NODES, N_SEG, SEG_W).transpose(0, 2, 1)
    xp = xp.reshape(N_NODES, D_FEAT)

    gate4 = gate
    sd = jnp.stack([src_idx.reshape(NW, NCH, CH),
                    dst_idx.reshape(NW, NCH, CH)], axis=2)
    partials = _sc_scatter(xp, gate4, sd)
    outp = _combine(partials)

    out = outp.reshape(N_NODES, SEG_W, N_SEG).transpose(0, 2, 1)
    return out.reshape(N_NODES, D_FEAT)
